# Initial kernel scaffold; baseline (speedup 1.0000x reference)
import jax
import jax.numpy as jnp
from jax.experimental import pallas as pl


def kernel(x, edge_index, W1, b1, W2, b2):
    raise NotImplementedError("write your pallas kernel here")


# speedup vs baseline: 16.7873x; 16.7873x over previous
import functools

import jax
import jax.numpy as jnp
from jax import lax
from jax.experimental import pallas as pl
from jax.experimental.pallas import tpu as pltpu
from jax.experimental.pallas import tpu_sc as plsc

N = 10000
NPAD = 10240
E = 320000
IN_DIM = 128
HID_DIM = 256
OUT_DIM = 128

NC = 2
NS = 16
BLK = 125
EROWS = E // BLK
RPT = NPAD // NS

_mesh = plsc.VectorSubcoreMesh(core_axis_name="c", subcore_axis_name="s")
f32 = jnp.float32


def _sds(shape, dtype=f32):
    return jax.ShapeDtypeStruct(shape, dtype)


def _deg_run(dst_rows, ones_hbm, zeros_hbm, out, idx_v, ones_v, acc, s, row0):
    nblk = EROWS // (NC * NS)
    pltpu.sync_copy(zeros_hbm, acc.at[pl.ds(s * RPT, RPT)])
    pltpu.sync_copy(ones_hbm, ones_v)
    pltpu.sync_copy(dst_rows.at[pl.ds(row0, nblk)], idx_v)
    plsc.subcore_barrier()

    def blk(j, carry):
        pltpu.sync_copy(ones_v, acc.at[idx_v.at[j]], add=True)
        return carry

    lax.fori_loop(0, nblk, blk, 0)
    plsc.subcore_barrier()
    sl = pl.ds(s * RPT, RPT)
    pltpu.sync_copy(acc.at[sl], out.at[sl])


def _deg_body(dst_rows, ones_hbm, zeros_hbm, deg0, deg1, idx_v, ones_v, acc):
    c = lax.axis_index("c")
    s = lax.axis_index("s")
    nblk = EROWS // (NC * NS)

    @pl.when(c == 0)
    def _():
        _deg_run(dst_rows, ones_hbm, zeros_hbm, deg0, idx_v, ones_v, acc,
                 s, s * nblk)

    @pl.when(c == 1)
    def _():
        _deg_run(dst_rows, ones_hbm, zeros_hbm, deg1, idx_v, ones_v, acc,
                 s, (NS + s) * nblk)


_deg_kernel = functools.partial(
    pl.kernel,
    out_type=(_sds((NPAD, 128)), _sds((NPAD, 128))),
    mesh=_mesh,
    scratch_types=[
        pltpu.VMEM((EROWS // (NC * NS), BLK), jnp.int32),
        pltpu.VMEM((BLK, 128), f32),
        pltpu.VMEM_SHARED((NPAD, 128), f32),
    ],
)(_deg_body)


CH = 40


def _agg_run(table, out, src_rows, dst_rows, zeros_hbm,
             idx_s, idx_d, rows, acc, sem, s, row0, nblk):
    pltpu.sync_copy(zeros_hbm, acc.at[pl.ds(s * RPT, RPT)])
    plsc.subcore_barrier()

    def chunk(ci, carry):
        r0 = pl.multiple_of(row0 + ci * CH, 8)
        pltpu.sync_copy(src_rows.at[pl.ds(r0, CH)], idx_s)
        pltpu.sync_copy(dst_rows.at[pl.ds(r0, CH)], idx_d)

        def blk(j, carry2):
            pltpu.async_copy(table.at[idx_s.at[j]], rows, sem).wait()
            pltpu.sync_copy(rows, acc.at[idx_d.at[j]], add=True)
            return carry2

        lax.fori_loop(0, CH, blk, 0)
        return carry

    lax.fori_loop(0, nblk // CH, chunk, 0)
    plsc.subcore_barrier()
    sl = pl.ds(s * RPT, RPT)
    pltpu.sync_copy(acc.at[sl], out.at[sl])


def _agg1_body(ya, yb, src_rows, dst_rows, zeros_hbm, za, zb,
               idx_s, idx_d, rows, acc, sem):
    c = lax.axis_index("c")
    s = lax.axis_index("s")
    nblk = EROWS // NS
    row0 = s * nblk

    @pl.when(c == 0)
    def _():
        _agg_run(ya, za, src_rows, dst_rows, zeros_hbm,
                 idx_s, idx_d, rows, acc, sem, s, row0, nblk)

    @pl.when(c == 1)
    def _():
        _agg_run(yb, zb, src_rows, dst_rows, zeros_hbm,
                 idx_s, idx_d, rows, acc, sem, s, row0, nblk)


_agg1_kernel = functools.partial(
    pl.kernel,
    out_type=(_sds((NPAD, 128)), _sds((NPAD, 128))),
    mesh=_mesh,
    scratch_types=[
        pltpu.VMEM((CH, BLK), jnp.int32),
        pltpu.VMEM((CH, BLK), jnp.int32),
        pltpu.VMEM((BLK, 128), f32),
        pltpu.VMEM_SHARED((NPAD, 128), f32),
        pltpu.SemaphoreType.DMA,
    ],
)(_agg1_body)


def _agg2_body(y2, src_rows, dst_rows, zeros_hbm, z0, z1,
               idx_s, idx_d, rows, acc, sem):
    c = lax.axis_index("c")
    s = lax.axis_index("s")
    nblk = EROWS // (NC * NS)
    row0 = (c * NS + s) * nblk

    @pl.when(c == 0)
    def _():
        _agg_run(y2, z0, src_rows, dst_rows, zeros_hbm,
                 idx_s, idx_d, rows, acc, sem, s, row0, nblk)

    @pl.when(c == 1)
    def _():
        _agg_run(y2, z1, src_rows, dst_rows, zeros_hbm,
                 idx_s, idx_d, rows, acc, sem, s, row0, nblk)


_agg2_kernel = functools.partial(
    pl.kernel,
    out_type=(_sds((NPAD, 128)), _sds((NPAD, 128))),
    mesh=_mesh,
    scratch_types=[
        pltpu.VMEM((CH, BLK), jnp.int32),
        pltpu.VMEM((CH, BLK), jnp.int32),
        pltpu.VMEM((BLK, 128), f32),
        pltpu.VMEM_SHARED((NPAD, 128), f32),
        pltpu.SemaphoreType.DMA,
    ],
)(_agg2_body)


RB = 1000
GRID = N // RB


def _dinv(d0_ref, d1_ref):
    deg = d0_ref[:, :1] + d1_ref[:, :1] + 1.0
    return lax.rsqrt(deg)


def _tc1_body(x_ref, d0_ref, d1_ref, w1_ref, ya_ref, yb_ref):
    dinv = _dinv(d0_ref, d1_ref)
    h = jnp.dot(x_ref[...], w1_ref[...], preferred_element_type=f32)
    y = h * dinv
    ya_ref[...] = y[:, :128]
    yb_ref[...] = y[:, 128:]


def _tc2_body(za_ref, zb_ref, ya_ref, yb_ref, d0_ref, d1_ref, w2_ref, b1_ref,
              y2_ref):
    dinv = _dinv(d0_ref, d1_ref)
    b1 = b1_ref[...]
    ha = jnp.maximum(dinv * (za_ref[...] + ya_ref[...]) + b1[:, :128], 0.0)
    hb = jnp.maximum(dinv * (zb_ref[...] + yb_ref[...]) + b1[:, 128:], 0.0)
    h1 = jnp.concatenate([ha, hb], axis=1)
    y2_ref[...] = dinv * jnp.dot(h1, w2_ref[...], preferred_element_type=f32)


def _tc3_body(z0_ref, z1_ref, y2_ref, d0_ref, d1_ref, b2_ref, out_ref):
    dinv = _dinv(d0_ref, d1_ref)
    out_ref[...] = dinv * (z0_ref[...] + z1_ref[...] + y2_ref[...]) + b2_ref[...]


def _row_spec(cols):
    return pl.BlockSpec((RB, cols), lambda i: (i, 0))


def _full_spec(r, c):
    return pl.BlockSpec((r, c), lambda i: (0, 0))


def _tc1(x, d0, d1, W1):
    return pl.pallas_call(
        _tc1_body,
        grid=(GRID,),
        in_specs=[_row_spec(IN_DIM), _row_spec(128), _row_spec(128),
                  _full_spec(IN_DIM, HID_DIM)],
        out_specs=(_row_spec(128), _row_spec(128)),
        out_shape=(_sds((NPAD, 128)), _sds((NPAD, 128))),
    )(x, d0, d1, W1)


def _tc2(za, zb, ya, yb, d0, d1, W2, b1):
    return pl.pallas_call(
        _tc2_body,
        grid=(GRID,),
        in_specs=[_row_spec(128), _row_spec(128), _row_spec(128),
                  _row_spec(128), _row_spec(128), _row_spec(128),
                  _full_spec(HID_DIM, OUT_DIM), _full_spec(1, HID_DIM)],
        out_specs=_row_spec(OUT_DIM),
        out_shape=_sds((NPAD, OUT_DIM)),
    )(za, zb, ya, yb, d0, d1, W2, b1)


def _tc3(z0, z1, y2, d0, d1, b2):
    return pl.pallas_call(
        _tc3_body,
        grid=(GRID,),
        in_specs=[_row_spec(128), _row_spec(128), _row_spec(128),
                  _row_spec(128), _row_spec(128), _full_spec(1, OUT_DIM)],
        out_specs=_row_spec(OUT_DIM),
        out_shape=_sds((N, OUT_DIM)),
    )(z0, z1, y2, d0, d1, b2)


def kernel(x, edge_index, W1, b1, W2, b2):
    src_rows = edge_index[0].reshape(EROWS, BLK)
    dst_rows = edge_index[1].reshape(EROWS, BLK)
    ones128 = jnp.ones((BLK, 128), f32)
    zeros128 = jnp.zeros((RPT, 128), f32)
    b1r = b1.reshape(1, HID_DIM)
    b2r = b2.reshape(1, OUT_DIM)

    d0, d1 = _deg_kernel(dst_rows, ones128, zeros128)
    ya, yb = _tc1(x, d0, d1, W1)
    za, zb = _agg1_kernel(ya, yb, src_rows, dst_rows, zeros128)
    y2 = _tc2(za, zb, ya, yb, d0, d1, W2, b1r)
    z0, z1 = _agg2_kernel(y2, src_rows, dst_rows, zeros128)
    return _tc3(z0, z1, y2, d0, d1, b2r)


# speedup vs baseline: 24.4042x; 1.4537x over previous
import functools

import jax
import jax.numpy as jnp
from jax import lax
from jax.experimental import pallas as pl
from jax.experimental.pallas import tpu as pltpu
from jax.experimental.pallas import tpu_sc as plsc

N = 10000
NPAD = 10240
E = 320000
IN_DIM = 128
HID_DIM = 256
OUT_DIM = 128

NC = 2
NS = 16
BLK = 125
EROWS = E // BLK
RPT = NPAD // NS

_mesh = plsc.VectorSubcoreMesh(core_axis_name="c", subcore_axis_name="s")
f32 = jnp.float32


def _sds(shape, dtype=f32):
    return jax.ShapeDtypeStruct(shape, dtype)


def _deg_run(dst_rows, ones_hbm, zeros_hbm, out, idx_v, ones_v, acc, sem,
             s, row0):
    nblk = EROWS // (NC * NS)
    pltpu.sync_copy(zeros_hbm, acc.at[pl.ds(s * RPT, RPT)])
    pltpu.sync_copy(ones_hbm, ones_v)
    pltpu.sync_copy(dst_rows.at[pl.ds(row0, nblk)], idx_v)
    plsc.subcore_barrier()
    pltpu.async_copy(ones_v, acc.at[idx_v.at[0]], sem, add=True)

    def blk(j, carry):
        pltpu.async_copy(ones_v, acc.at[idx_v.at[j]], sem, add=True)
        pltpu.make_async_copy(ones_v, acc.at[idx_v.at[j - 1]], sem).wait()
        return carry

    lax.fori_loop(1, nblk, blk, 0)
    pltpu.make_async_copy(ones_v, acc.at[idx_v.at[nblk - 1]], sem).wait()
    plsc.subcore_barrier()
    sl = pl.ds(s * RPT, RPT)
    pltpu.sync_copy(acc.at[sl], out.at[sl])


def _deg_body(dst_rows, ones_hbm, zeros_hbm, deg0, deg1, idx_v, ones_v, acc,
              sem):
    c = lax.axis_index("c")
    s = lax.axis_index("s")
    nblk = EROWS // (NC * NS)

    @pl.when(c == 0)
    def _():
        _deg_run(dst_rows, ones_hbm, zeros_hbm, deg0, idx_v, ones_v, acc,
                 sem, s, s * nblk)

    @pl.when(c == 1)
    def _():
        _deg_run(dst_rows, ones_hbm, zeros_hbm, deg1, idx_v, ones_v, acc,
                 sem, s, (NS + s) * nblk)


_deg_kernel = functools.partial(
    pl.kernel,
    out_type=(_sds((NPAD, 128)), _sds((NPAD, 128))),
    mesh=_mesh,
    scratch_types=[
        pltpu.VMEM((EROWS // (NC * NS), BLK), jnp.int32),
        pltpu.VMEM((BLK, 128), f32),
        pltpu.VMEM_SHARED((NPAD, 128), f32),
        pltpu.SemaphoreType.DMA,
    ],
)(_deg_body)


CH = 40


def _agg_run(table, out, src_rows, dst_rows, zeros_hbm,
             idx_s, idx_d, rows0, rows1, acc, sem, s, row0, nblk):
    pltpu.sync_copy(zeros_hbm, acc.at[pl.ds(s * RPT, RPT)])
    plsc.subcore_barrier()

    def _gat(j, buf):
        pltpu.async_copy(table.at[idx_s.at[j]], buf, sem)

    def _wait(j, buf):
        pltpu.make_async_copy(table.at[idx_s.at[j]], buf, sem).wait()

    def _sca(j, buf):
        pltpu.sync_copy(buf, acc.at[idx_d.at[j]], add=True)

    def chunk(ci, carry):
        r0 = pl.multiple_of(row0 + ci * CH, 8)
        pltpu.sync_copy(src_rows.at[pl.ds(r0, CH)], idx_s)
        pltpu.sync_copy(dst_rows.at[pl.ds(r0, CH)], idx_d)
        _gat(0, rows0)

        def pair(j2, carry2):
            j = j2 * 2
            _gat(j + 1, rows1)
            _wait(j, rows0)
            _sca(j, rows0)
            _gat(j + 2, rows0)
            _wait(j + 1, rows1)
            _sca(j + 1, rows1)
            return carry2

        lax.fori_loop(0, CH // 2 - 1, pair, 0)
        _gat(CH - 1, rows1)
        _wait(CH - 2, rows0)
        _sca(CH - 2, rows0)
        _wait(CH - 1, rows1)
        _sca(CH - 1, rows1)
        return carry

    lax.fori_loop(0, nblk // CH, chunk, 0)
    plsc.subcore_barrier()
    sl = pl.ds(s * RPT, RPT)
    pltpu.sync_copy(acc.at[sl], out.at[sl])


def _agg1_body(ya, yb, src_rows, dst_rows, zeros_hbm, za, zb,
               idx_s, idx_d, rows0, rows1, acc, sem):
    c = lax.axis_index("c")
    s = lax.axis_index("s")
    nblk = EROWS // NS
    row0 = s * nblk

    @pl.when(c == 0)
    def _():
        _agg_run(ya, za, src_rows, dst_rows, zeros_hbm,
                 idx_s, idx_d, rows0, rows1, acc, sem, s, row0, nblk)

    @pl.when(c == 1)
    def _():
        _agg_run(yb, zb, src_rows, dst_rows, zeros_hbm,
                 idx_s, idx_d, rows0, rows1, acc, sem, s, row0, nblk)


_agg1_kernel = functools.partial(
    pl.kernel,
    out_type=(_sds((NPAD, 128)), _sds((NPAD, 128))),
    mesh=_mesh,
    scratch_types=[
        pltpu.VMEM((CH, BLK), jnp.int32),
        pltpu.VMEM((CH, BLK), jnp.int32),
        pltpu.VMEM((BLK, 128), f32),
        pltpu.VMEM((BLK, 128), f32),
        pltpu.VMEM_SHARED((NPAD, 128), f32),
        pltpu.SemaphoreType.DMA,
    ],
)(_agg1_body)


def _agg2_body(y2, src_rows, dst_rows, zeros_hbm, z0, z1,
               idx_s, idx_d, rows0, rows1, acc, sem):
    c = lax.axis_index("c")
    s = lax.axis_index("s")
    nblk = EROWS // (NC * NS)
    row0 = (c * NS + s) * nblk

    @pl.when(c == 0)
    def _():
        _agg_run(y2, z0, src_rows, dst_rows, zeros_hbm,
                 idx_s, idx_d, rows0, rows1, acc, sem, s, row0, nblk)

    @pl.when(c == 1)
    def _():
        _agg_run(y2, z1, src_rows, dst_rows, zeros_hbm,
                 idx_s, idx_d, rows0, rows1, acc, sem, s, row0, nblk)


_agg2_kernel = functools.partial(
    pl.kernel,
    out_type=(_sds((NPAD, 128)), _sds((NPAD, 128))),
    mesh=_mesh,
    scratch_types=[
        pltpu.VMEM((CH, BLK), jnp.int32),
        pltpu.VMEM((CH, BLK), jnp.int32),
        pltpu.VMEM((BLK, 128), f32),
        pltpu.VMEM((BLK, 128), f32),
        pltpu.VMEM_SHARED((NPAD, 128), f32),
        pltpu.SemaphoreType.DMA,
    ],
)(_agg2_body)


RB = 1000
GRID = N // RB


def _dinv(d0_ref, d1_ref):
    deg = d0_ref[:, :1] + d1_ref[:, :1] + 1.0
    return lax.rsqrt(deg)


def _tc1_body(x_ref, d0_ref, d1_ref, w1_ref, ya_ref, yb_ref):
    dinv = _dinv(d0_ref, d1_ref)
    h = jnp.dot(x_ref[...], w1_ref[...], preferred_element_type=f32)
    y = h * dinv
    ya_ref[...] = y[:, :128]
    yb_ref[...] = y[:, 128:]


def _tc2_body(za_ref, zb_ref, ya_ref, yb_ref, d0_ref, d1_ref, w2_ref, b1_ref,
              y2_ref):
    dinv = _dinv(d0_ref, d1_ref)
    b1 = b1_ref[...]
    ha = jnp.maximum(dinv * (za_ref[...] + ya_ref[...]) + b1[:, :128], 0.0)
    hb = jnp.maximum(dinv * (zb_ref[...] + yb_ref[...]) + b1[:, 128:], 0.0)
    h1 = jnp.concatenate([ha, hb], axis=1)
    y2_ref[...] = dinv * jnp.dot(h1, w2_ref[...], preferred_element_type=f32)


def _tc3_body(z0_ref, z1_ref, y2_ref, d0_ref, d1_ref, b2_ref, out_ref):
    dinv = _dinv(d0_ref, d1_ref)
    out_ref[...] = dinv * (z0_ref[...] + z1_ref[...] + y2_ref[...]) + b2_ref[...]


def _row_spec(cols):
    return pl.BlockSpec((RB, cols), lambda i: (i, 0))


def _full_spec(r, c):
    return pl.BlockSpec((r, c), lambda i: (0, 0))


def _tc1(x, d0, d1, W1):
    return pl.pallas_call(
        _tc1_body,
        grid=(GRID,),
        in_specs=[_row_spec(IN_DIM), _row_spec(128), _row_spec(128),
                  _full_spec(IN_DIM, HID_DIM)],
        out_specs=(_row_spec(128), _row_spec(128)),
        out_shape=(_sds((NPAD, 128)), _sds((NPAD, 128))),
    )(x, d0, d1, W1)


def _tc2(za, zb, ya, yb, d0, d1, W2, b1):
    return pl.pallas_call(
        _tc2_body,
        grid=(GRID,),
        in_specs=[_row_spec(128), _row_spec(128), _row_spec(128),
                  _row_spec(128), _row_spec(128), _row_spec(128),
                  _full_spec(HID_DIM, OUT_DIM), _full_spec(1, HID_DIM)],
        out_specs=_row_spec(OUT_DIM),
        out_shape=_sds((NPAD, OUT_DIM)),
    )(za, zb, ya, yb, d0, d1, W2, b1)


def _tc3(z0, z1, y2, d0, d1, b2):
    return pl.pallas_call(
        _tc3_body,
        grid=(GRID,),
        in_specs=[_row_spec(128), _row_spec(128), _row_spec(128),
                  _row_spec(128), _row_spec(128), _full_spec(1, OUT_DIM)],
        out_specs=_row_spec(OUT_DIM),
        out_shape=_sds((N, OUT_DIM)),
    )(z0, z1, y2, d0, d1, b2)


def kernel(x, edge_index, W1, b1, W2, b2):
    src_rows = edge_index[0].reshape(EROWS, BLK)
    dst_rows = edge_index[1].reshape(EROWS, BLK)
    ones128 = jnp.ones((BLK, 128), f32)
    zeros128 = jnp.zeros((RPT, 128), f32)
    b1r = b1.reshape(1, HID_DIM)
    b2r = b2.reshape(1, OUT_DIM)

    d0, d1 = _deg_kernel(dst_rows, ones128, zeros128)
    ya, yb = _tc1(x, d0, d1, W1)
    za, zb = _agg1_kernel(ya, yb, src_rows, dst_rows, zeros128)
    y2 = _tc2(za, zb, ya, yb, d0, d1, W2, b1r)
    z0, z1 = _agg2_kernel(y2, src_rows, dst_rows, zeros128)
    return _tc3(z0, z1, y2, d0, d1, b2r)


# speedup vs baseline: 24.5890x; 1.0076x over previous
import functools

import jax
import jax.numpy as jnp
from jax import lax
from jax.experimental import pallas as pl
from jax.experimental.pallas import tpu as pltpu
from jax.experimental.pallas import tpu_sc as plsc

N = 10000
NPAD = 10240
E = 320000
IN_DIM = 128
HID_DIM = 256
OUT_DIM = 128

NC = 2
NS = 16
BLK = 125
EROWS = E // BLK
ABLK = 50
AROWS = E // ABLK
RPT = NPAD // NS

_mesh = plsc.VectorSubcoreMesh(core_axis_name="c", subcore_axis_name="s")
f32 = jnp.float32


def _sds(shape, dtype=f32):
    return jax.ShapeDtypeStruct(shape, dtype)


def _deg_run(dst_rows, ones_hbm, zeros_hbm, out, idx_v, ones_v, acc, sem,
             s, row0):
    nblk = EROWS // (NC * NS)
    pltpu.sync_copy(zeros_hbm, acc.at[pl.ds(s * RPT, RPT)])
    pltpu.sync_copy(ones_hbm, ones_v)
    pltpu.sync_copy(dst_rows.at[pl.ds(row0, nblk)], idx_v)
    plsc.subcore_barrier()
    pltpu.async_copy(ones_v, acc.at[idx_v.at[0]], sem, add=True)

    def blk(j, carry):
        pltpu.async_copy(ones_v, acc.at[idx_v.at[j]], sem, add=True)
        pltpu.make_async_copy(ones_v, acc.at[idx_v.at[j - 1]], sem).wait()
        return carry

    lax.fori_loop(1, nblk, blk, 0)
    pltpu.make_async_copy(ones_v, acc.at[idx_v.at[nblk - 1]], sem).wait()
    plsc.subcore_barrier()
    sl = pl.ds(s * RPT, RPT)
    pltpu.sync_copy(acc.at[sl], out.at[sl])


def _deg_body(dst_rows, ones_hbm, zeros_hbm, deg0, deg1, idx_v, ones_v, acc,
              sem):
    c = lax.axis_index("c")
    s = lax.axis_index("s")
    nblk = EROWS // (NC * NS)

    @pl.when(c == 0)
    def _():
        _deg_run(dst_rows, ones_hbm, zeros_hbm, deg0, idx_v, ones_v, acc,
                 sem, s, s * nblk)

    @pl.when(c == 1)
    def _():
        _deg_run(dst_rows, ones_hbm, zeros_hbm, deg1, idx_v, ones_v, acc,
                 sem, s, (NS + s) * nblk)


_deg_kernel = functools.partial(
    pl.kernel,
    out_type=(_sds((NPAD, 128)), _sds((NPAD, 128))),
    mesh=_mesh,
    scratch_types=[
        pltpu.VMEM((EROWS // (NC * NS), BLK), jnp.int32),
        pltpu.VMEM((BLK, 128), f32),
        pltpu.VMEM_SHARED((NPAD, 128), f32),
        pltpu.SemaphoreType.DMA,
    ],
)(_deg_body)


ACH = 40


def _agg_run(table, out, src_rows, dst_rows, zeros_hbm,
             idx_s, idx_d, bufs, acc, gsem, ssem, s, row0, nblk):
    pltpu.sync_copy(zeros_hbm, acc.at[pl.ds(s * RPT, RPT)])
    plsc.subcore_barrier()

    def _gat(j, buf):
        pltpu.async_copy(table.at[idx_s.at[j]], buf, gsem)

    def _gwait(j, buf):
        pltpu.make_async_copy(table.at[idx_s.at[j]], buf, gsem).wait()

    def _sca(j, buf):
        pltpu.async_copy(buf, acc.at[idx_d.at[j]], ssem, add=True)

    def _swait(j, buf):
        pltpu.make_async_copy(buf, acc.at[idx_d.at[j]], ssem).wait()

    def chunk(ci, carry):
        r0 = pl.multiple_of(row0 + ci * ACH, 8)
        pltpu.sync_copy(src_rows.at[pl.ds(r0, ACH)], idx_s)
        pltpu.sync_copy(dst_rows.at[pl.ds(r0, ACH)], idx_d)
        for b in range(4):
            _gat(b, bufs[b])

        def group(m, carry2):
            j = m * 4
            _gwait(j, bufs[0])
            _sca(j, bufs[0])
            _gwait(j + 1, bufs[1])
            _sca(j + 1, bufs[1])
            _swait(j, bufs[0])
            _gat(j + 4, bufs[0])
            _gwait(j + 2, bufs[2])
            _sca(j + 2, bufs[2])
            _swait(j + 1, bufs[1])
            _gat(j + 5, bufs[1])
            _gwait(j + 3, bufs[3])
            _sca(j + 3, bufs[3])
            _swait(j + 2, bufs[2])
            _gat(j + 6, bufs[2])
            _swait(j + 3, bufs[3])
            _gat(j + 7, bufs[3])
            return carry2

        lax.fori_loop(0, ACH // 4 - 1, group, 0)
        je = ACH - 4
        for b in range(4):
            _gwait(je + b, bufs[b])
            _sca(je + b, bufs[b])
        for b in range(4):
            _swait(je + b, bufs[b])
        return carry

    lax.fori_loop(0, nblk // ACH, chunk, 0)
    plsc.subcore_barrier()
    sl = pl.ds(s * RPT, RPT)
    pltpu.sync_copy(acc.at[sl], out.at[sl])


def _agg1_body(ya, yb, src_rows, dst_rows, zeros_hbm, za, zb,
               idx_s, idx_d, r0b, r1b, r2b, r3b, acc, gsem, ssem):
    c = lax.axis_index("c")
    s = lax.axis_index("s")
    bufs = (r0b, r1b, r2b, r3b)
    nblk = AROWS // NS
    row0 = s * nblk

    @pl.when(c == 0)
    def _():
        _agg_run(ya, za, src_rows, dst_rows, zeros_hbm,
                 idx_s, idx_d, bufs, acc, gsem, ssem, s, row0, nblk)

    @pl.when(c == 1)
    def _():
        _agg_run(yb, zb, src_rows, dst_rows, zeros_hbm,
                 idx_s, idx_d, bufs, acc, gsem, ssem, s, row0, nblk)


_agg1_kernel = functools.partial(
    pl.kernel,
    out_type=(_sds((NPAD, 128)), _sds((NPAD, 128))),
    mesh=_mesh,
    scratch_types=[
        pltpu.VMEM((ACH, ABLK), jnp.int32),
        pltpu.VMEM((ACH, ABLK), jnp.int32),
        pltpu.VMEM((ABLK, 128), f32),
        pltpu.VMEM((ABLK, 128), f32),
        pltpu.VMEM((ABLK, 128), f32),
        pltpu.VMEM((ABLK, 128), f32),
        pltpu.VMEM_SHARED((NPAD, 128), f32),
        pltpu.SemaphoreType.DMA,
        pltpu.SemaphoreType.DMA,
    ],
)(_agg1_body)


def _agg2_body(y2, src_rows, dst_rows, zeros_hbm, z0, z1,
               idx_s, idx_d, r0b, r1b, r2b, r3b, acc, gsem, ssem):
    c = lax.axis_index("c")
    s = lax.axis_index("s")
    bufs = (r0b, r1b, r2b, r3b)
    nblk = AROWS // (NC * NS)
    row0 = (c * NS + s) * nblk

    @pl.when(c == 0)
    def _():
        _agg_run(y2, z0, src_rows, dst_rows, zeros_hbm,
                 idx_s, idx_d, bufs, acc, gsem, ssem, s, row0, nblk)

    @pl.when(c == 1)
    def _():
        _agg_run(y2, z1, src_rows, dst_rows, zeros_hbm,
                 idx_s, idx_d, bufs, acc, gsem, ssem, s, row0, nblk)


_agg2_kernel = functools.partial(
    pl.kernel,
    out_type=(_sds((NPAD, 128)), _sds((NPAD, 128))),
    mesh=_mesh,
    scratch_types=[
        pltpu.VMEM((ACH, ABLK), jnp.int32),
        pltpu.VMEM((ACH, ABLK), jnp.int32),
        pltpu.VMEM((ABLK, 128), f32),
        pltpu.VMEM((ABLK, 128), f32),
        pltpu.VMEM((ABLK, 128), f32),
        pltpu.VMEM((ABLK, 128), f32),
        pltpu.VMEM_SHARED((NPAD, 128), f32),
        pltpu.SemaphoreType.DMA,
        pltpu.SemaphoreType.DMA,
    ],
)(_agg2_body)


RB = 1000
GRID = N // RB


def _dinv(d0_ref, d1_ref):
    deg = d0_ref[:, :1] + d1_ref[:, :1] + 1.0
    return lax.rsqrt(deg)


def _tc1_body(x_ref, d0_ref, d1_ref, w1_ref, ya_ref, yb_ref):
    dinv = _dinv(d0_ref, d1_ref)
    h = jnp.dot(x_ref[...], w1_ref[...], preferred_element_type=f32)
    y = h * dinv
    ya_ref[...] = y[:, :128]
    yb_ref[...] = y[:, 128:]


def _tc2_body(za_ref, zb_ref, ya_ref, yb_ref, d0_ref, d1_ref, w2_ref, b1_ref,
              y2_ref):
    dinv = _dinv(d0_ref, d1_ref)
    b1 = b1_ref[...]
    ha = jnp.maximum(dinv * (za_ref[...] + ya_ref[...]) + b1[:, :128], 0.0)
    hb = jnp.maximum(dinv * (zb_ref[...] + yb_ref[...]) + b1[:, 128:], 0.0)
    h1 = jnp.concatenate([ha, hb], axis=1)
    y2_ref[...] = dinv * jnp.dot(h1, w2_ref[...], preferred_element_type=f32)


def _tc3_body(z0_ref, z1_ref, y2_ref, d0_ref, d1_ref, b2_ref, out_ref):
    dinv = _dinv(d0_ref, d1_ref)
    out_ref[...] = dinv * (z0_ref[...] + z1_ref[...] + y2_ref[...]) + b2_ref[...]


def _row_spec(cols):
    return pl.BlockSpec((RB, cols), lambda i: (i, 0))


def _full_spec(r, c):
    return pl.BlockSpec((r, c), lambda i: (0, 0))


def _tc1(x, d0, d1, W1):
    return pl.pallas_call(
        _tc1_body,
        grid=(GRID,),
        in_specs=[_row_spec(IN_DIM), _row_spec(128), _row_spec(128),
                  _full_spec(IN_DIM, HID_DIM)],
        out_specs=(_row_spec(128), _row_spec(128)),
        out_shape=(_sds((NPAD, 128)), _sds((NPAD, 128))),
    )(x, d0, d1, W1)


def _tc2(za, zb, ya, yb, d0, d1, W2, b1):
    return pl.pallas_call(
        _tc2_body,
        grid=(GRID,),
        in_specs=[_row_spec(128), _row_spec(128), _row_spec(128),
                  _row_spec(128), _row_spec(128), _row_spec(128),
                  _full_spec(HID_DIM, OUT_DIM), _full_spec(1, HID_DIM)],
        out_specs=_row_spec(OUT_DIM),
        out_shape=_sds((NPAD, OUT_DIM)),
    )(za, zb, ya, yb, d0, d1, W2, b1)


def _tc3(z0, z1, y2, d0, d1, b2):
    return pl.pallas_call(
        _tc3_body,
        grid=(GRID,),
        in_specs=[_row_spec(128), _row_spec(128), _row_spec(128),
                  _row_spec(128), _row_spec(128), _full_spec(1, OUT_DIM)],
        out_specs=_row_spec(OUT_DIM),
        out_shape=_sds((N, OUT_DIM)),
    )(z0, z1, y2, d0, d1, b2)


def kernel(x, edge_index, W1, b1, W2, b2):
    src_rows = edge_index[0].reshape(EROWS, BLK)
    dst_rows = edge_index[1].reshape(EROWS, BLK)
    src_rows_a = edge_index[0].reshape(AROWS, ABLK)
    dst_rows_a = edge_index[1].reshape(AROWS, ABLK)
    ones128 = jnp.ones((BLK, 128), f32)
    zeros128 = jnp.zeros((RPT, 128), f32)
    b1r = b1.reshape(1, HID_DIM)
    b2r = b2.reshape(1, OUT_DIM)

    d0, d1 = _deg_kernel(dst_rows, ones128, zeros128)
    ya, yb = _tc1(x, d0, d1, W1)
    za, zb = _agg1_kernel(ya, yb, src_rows_a, dst_rows_a, zeros128)
    y2 = _tc2(za, zb, ya, yb, d0, d1, W2, b1r)
    z0, z1 = _agg2_kernel(y2, src_rows_a, dst_rows_a, zeros128)
    return _tc3(z0, z1, y2, d0, d1, b2r)


# speedup vs baseline: 25.1143x; 1.0214x over previous
import functools

import jax
import jax.numpy as jnp
from jax import lax
from jax.experimental import pallas as pl
from jax.experimental.pallas import tpu as pltpu
from jax.experimental.pallas import tpu_sc as plsc

N = 10000
NPAD = 10240
E = 320000
IN_DIM = 128
HID_DIM = 256
OUT_DIM = 128

NC = 2
NS = 16
BLK = 125
EROWS = E // BLK
ABLK = 50
AROWS = E // ABLK
RPT = NPAD // NS

_mesh = plsc.VectorSubcoreMesh(core_axis_name="c", subcore_axis_name="s")
f32 = jnp.float32


def _sds(shape, dtype=f32):
    return jax.ShapeDtypeStruct(shape, dtype)


def _deg_run(dst_rows, ones_hbm, zeros_hbm, out, idx_v, ones_v, acc, sem,
             s, row0):
    nblk = EROWS // (NC * NS)
    pltpu.sync_copy(zeros_hbm, acc.at[pl.ds(s * RPT, RPT)])
    pltpu.sync_copy(ones_hbm, ones_v)
    pltpu.sync_copy(dst_rows.at[pl.ds(row0, nblk)], idx_v)
    plsc.subcore_barrier()
    DEPTH = 8
    for j in range(DEPTH):
        pltpu.async_copy(ones_v, acc.at[idx_v.at[j]], sem, add=True)

    def blk(j, carry):
        pltpu.async_copy(ones_v, acc.at[idx_v.at[j]], sem, add=True)
        pltpu.make_async_copy(ones_v, acc.at[idx_v.at[j - DEPTH]], sem).wait()
        return carry

    lax.fori_loop(DEPTH, nblk, blk, 0)

    def drain(j, carry):
        pltpu.make_async_copy(ones_v, acc.at[idx_v.at[j]], sem).wait()
        return carry

    lax.fori_loop(nblk - DEPTH, nblk, drain, 0)
    plsc.subcore_barrier()
    sl = pl.ds(s * RPT, RPT)
    pltpu.sync_copy(acc.at[sl], out.at[sl])


def _deg_body(dst_rows, ones_hbm, zeros_hbm, deg0, deg1, idx_v, ones_v, acc,
              sem):
    c = lax.axis_index("c")
    s = lax.axis_index("s")
    nblk = EROWS // (NC * NS)

    @pl.when(c == 0)
    def _():
        _deg_run(dst_rows, ones_hbm, zeros_hbm, deg0, idx_v, ones_v, acc,
                 sem, s, s * nblk)

    @pl.when(c == 1)
    def _():
        _deg_run(dst_rows, ones_hbm, zeros_hbm, deg1, idx_v, ones_v, acc,
                 sem, s, (NS + s) * nblk)


_deg_kernel = functools.partial(
    pl.kernel,
    out_type=(_sds((NPAD, 128)), _sds((NPAD, 128))),
    mesh=_mesh,
    scratch_types=[
        pltpu.VMEM((EROWS // (NC * NS), BLK), jnp.int32),
        pltpu.VMEM((BLK, 128), f32),
        pltpu.VMEM_SHARED((NPAD, 128), f32),
        pltpu.SemaphoreType.DMA,
    ],
)(_deg_body)


ACH1 = 80
ACH2 = 40


def _agg_run(table, out, src_rows, dst_rows, zeros_hbm,
             idx_s, idx_d, bufs, acc, gsem, ssem, s, row0, nblk, ach):
    pltpu.sync_copy(zeros_hbm, acc.at[pl.ds(s * RPT, RPT)])
    plsc.subcore_barrier()

    def _gat(j, buf):
        pltpu.async_copy(table.at[idx_s.at[j]], buf, gsem)

    def _gwait(j, buf):
        pltpu.make_async_copy(table.at[idx_s.at[j]], buf, gsem).wait()

    def _sca(j, buf):
        pltpu.async_copy(buf, acc.at[idx_d.at[j]], ssem, add=True)

    def _swait(j, buf):
        pltpu.make_async_copy(buf, acc.at[idx_d.at[j]], ssem).wait()

    def chunk(ci, carry):
        r0 = pl.multiple_of(row0 + ci * ach, 8)
        pltpu.sync_copy(src_rows.at[pl.ds(r0, ach)], idx_s)
        pltpu.sync_copy(dst_rows.at[pl.ds(r0, ach)], idx_d)
        for b in range(4):
            _gat(b, bufs[b])

        def group(m, carry2):
            j = m * 4
            _gwait(j, bufs[0])
            _sca(j, bufs[0])
            _gwait(j + 1, bufs[1])
            _sca(j + 1, bufs[1])
            _swait(j, bufs[0])
            _gat(j + 4, bufs[0])
            _gwait(j + 2, bufs[2])
            _sca(j + 2, bufs[2])
            _swait(j + 1, bufs[1])
            _gat(j + 5, bufs[1])
            _gwait(j + 3, bufs[3])
            _sca(j + 3, bufs[3])
            _swait(j + 2, bufs[2])
            _gat(j + 6, bufs[2])
            _swait(j + 3, bufs[3])
            _gat(j + 7, bufs[3])
            return carry2

        lax.fori_loop(0, ach // 4 - 1, group, 0)
        je = ach - 4
        for b in range(4):
            _gwait(je + b, bufs[b])
            _sca(je + b, bufs[b])
        for b in range(4):
            _swait(je + b, bufs[b])
        return carry

    lax.fori_loop(0, nblk // ach, chunk, 0)
    plsc.subcore_barrier()
    sl = pl.ds(s * RPT, RPT)
    pltpu.sync_copy(acc.at[sl], out.at[sl])


def _agg1_body(ya, yb, src_rows, dst_rows, zeros_hbm, za, zb,
               idx_s, idx_d, r0b, r1b, r2b, r3b, acc, gsem, ssem):
    c = lax.axis_index("c")
    s = lax.axis_index("s")
    bufs = (r0b, r1b, r2b, r3b)
    nblk = AROWS // NS
    row0 = s * nblk

    @pl.when(c == 0)
    def _():
        _agg_run(ya, za, src_rows, dst_rows, zeros_hbm,
                 idx_s, idx_d, bufs, acc, gsem, ssem, s, row0, nblk, ACH1)

    @pl.when(c == 1)
    def _():
        _agg_run(yb, zb, src_rows, dst_rows, zeros_hbm,
                 idx_s, idx_d, bufs, acc, gsem, ssem, s, row0, nblk, ACH1)


_agg1_kernel = functools.partial(
    pl.kernel,
    out_type=(_sds((NPAD, 128)), _sds((NPAD, 128))),
    mesh=_mesh,
    scratch_types=[
        pltpu.VMEM((ACH1, ABLK), jnp.int32),
        pltpu.VMEM((ACH1, ABLK), jnp.int32),
        pltpu.VMEM((ABLK, 128), f32),
        pltpu.VMEM((ABLK, 128), f32),
        pltpu.VMEM((ABLK, 128), f32),
        pltpu.VMEM((ABLK, 128), f32),
        pltpu.VMEM_SHARED((NPAD, 128), f32),
        pltpu.SemaphoreType.DMA,
        pltpu.SemaphoreType.DMA,
    ],
)(_agg1_body)


def _agg2_body(y2, src_rows, dst_rows, zeros_hbm, z0, z1,
               idx_s, idx_d, r0b, r1b, r2b, r3b, acc, gsem, ssem):
    c = lax.axis_index("c")
    s = lax.axis_index("s")
    bufs = (r0b, r1b, r2b, r3b)
    nblk = AROWS // (NC * NS)
    row0 = (c * NS + s) * nblk

    @pl.when(c == 0)
    def _():
        _agg_run(y2, z0, src_rows, dst_rows, zeros_hbm,
                 idx_s, idx_d, bufs, acc, gsem, ssem, s, row0, nblk, ACH2)

    @pl.when(c == 1)
    def _():
        _agg_run(y2, z1, src_rows, dst_rows, zeros_hbm,
                 idx_s, idx_d, bufs, acc, gsem, ssem, s, row0, nblk, ACH2)


_agg2_kernel = functools.partial(
    pl.kernel,
    out_type=(_sds((NPAD, 128)), _sds((NPAD, 128))),
    mesh=_mesh,
    scratch_types=[
        pltpu.VMEM((ACH2, ABLK), jnp.int32),
        pltpu.VMEM((ACH2, ABLK), jnp.int32),
        pltpu.VMEM((ABLK, 128), f32),
        pltpu.VMEM((ABLK, 128), f32),
        pltpu.VMEM((ABLK, 128), f32),
        pltpu.VMEM((ABLK, 128), f32),
        pltpu.VMEM_SHARED((NPAD, 128), f32),
        pltpu.SemaphoreType.DMA,
        pltpu.SemaphoreType.DMA,
    ],
)(_agg2_body)


RB = 1000
GRID = N // RB


def _dinv(d0_ref, d1_ref):
    deg = d0_ref[:, :1] + d1_ref[:, :1] + 1.0
    return lax.rsqrt(deg)


def _tc1_body(x_ref, d0_ref, d1_ref, w1_ref, ya_ref, yb_ref):
    dinv = _dinv(d0_ref, d1_ref)
    h = jnp.dot(x_ref[...], w1_ref[...], preferred_element_type=f32)
    y = h * dinv
    ya_ref[...] = y[:, :128]
    yb_ref[...] = y[:, 128:]


def _tc2_body(za_ref, zb_ref, ya_ref, yb_ref, d0_ref, d1_ref, w2_ref, b1_ref,
              y2_ref):
    dinv = _dinv(d0_ref, d1_ref)
    b1 = b1_ref[...]
    ha = jnp.maximum(dinv * (za_ref[...] + ya_ref[...]) + b1[:, :128], 0.0)
    hb = jnp.maximum(dinv * (zb_ref[...] + yb_ref[...]) + b1[:, 128:], 0.0)
    h1 = jnp.concatenate([ha, hb], axis=1)
    y2_ref[...] = dinv * jnp.dot(h1, w2_ref[...], preferred_element_type=f32)


def _tc3_body(z0_ref, z1_ref, y2_ref, d0_ref, d1_ref, b2_ref, out_ref):
    dinv = _dinv(d0_ref, d1_ref)
    out_ref[...] = dinv * (z0_ref[...] + z1_ref[...] + y2_ref[...]) + b2_ref[...]


def _row_spec(cols):
    return pl.BlockSpec((RB, cols), lambda i: (i, 0))


def _full_spec(r, c):
    return pl.BlockSpec((r, c), lambda i: (0, 0))


def _tc1(x, d0, d1, W1):
    return pl.pallas_call(
        _tc1_body,
        grid=(GRID,),
        in_specs=[_row_spec(IN_DIM), _row_spec(128), _row_spec(128),
                  _full_spec(IN_DIM, HID_DIM)],
        out_specs=(_row_spec(128), _row_spec(128)),
        out_shape=(_sds((NPAD, 128)), _sds((NPAD, 128))),
    )(x, d0, d1, W1)


def _tc2(za, zb, ya, yb, d0, d1, W2, b1):
    return pl.pallas_call(
        _tc2_body,
        grid=(GRID,),
        in_specs=[_row_spec(128), _row_spec(128), _row_spec(128),
                  _row_spec(128), _row_spec(128), _row_spec(128),
                  _full_spec(HID_DIM, OUT_DIM), _full_spec(1, HID_DIM)],
        out_specs=_row_spec(OUT_DIM),
        out_shape=_sds((NPAD, OUT_DIM)),
    )(za, zb, ya, yb, d0, d1, W2, b1)


def _tc3(z0, z1, y2, d0, d1, b2):
    return pl.pallas_call(
        _tc3_body,
        grid=(GRID,),
        in_specs=[_row_spec(128), _row_spec(128), _row_spec(128),
                  _row_spec(128), _row_spec(128), _full_spec(1, OUT_DIM)],
        out_specs=_row_spec(OUT_DIM),
        out_shape=_sds((N, OUT_DIM)),
    )(z0, z1, y2, d0, d1, b2)


def kernel(x, edge_index, W1, b1, W2, b2):
    src_rows = edge_index[0].reshape(EROWS, BLK)
    dst_rows = edge_index[1].reshape(EROWS, BLK)
    src_rows_a = edge_index[0].reshape(AROWS, ABLK)
    dst_rows_a = edge_index[1].reshape(AROWS, ABLK)
    ones128 = jnp.ones((BLK, 128), f32)
    zeros128 = jnp.zeros((RPT, 128), f32)
    b1r = b1.reshape(1, HID_DIM)
    b2r = b2.reshape(1, OUT_DIM)

    d0, d1 = _deg_kernel(dst_rows, ones128, zeros128)
    ya, yb = _tc1(x, d0, d1, W1)
    za, zb = _agg1_kernel(ya, yb, src_rows_a, dst_rows_a, zeros128)
    y2 = _tc2(za, zb, ya, yb, d0, d1, W2, b1r)
    z0, z1 = _agg2_kernel(y2, src_rows_a, dst_rows_a, zeros128)
    return _tc3(z0, z1, y2, d0, d1, b2r)
